# Initial kernel scaffold; baseline (speedup 1.0000x reference)
import jax
import jax.numpy as jnp
from jax.experimental import pallas as pl


def kernel(activation, hidden_state, sparsity_k, in_proj_w, in_proj_b, out_w, out_b, su_w1, su_b1, su_w2, su_b2, au_w1, au_b1, au_w2, au_b2, ln_g, ln_b):
    raise NotImplementedError("write your pallas kernel here")


# speedup vs baseline: 1.9984x; 1.9984x over previous
import functools

import jax
import jax.numpy as jnp
from jax import lax
from jax.experimental import pallas as pl
from jax.experimental.pallas import tpu as pltpu
from jax.experimental.pallas import tpu_sc as plsc

B, N, D, H = 16, 16384, 128, 4
K = 256
R = 128
C_ = 128
CHUNK = 512
NSC = 32


def _gelu(x):
    return 0.5 * x * (1.0 + lax.erf(x * 0.7071067811865476))


def _select_topk(a):
    bits = lax.bitcast_convert_type(a, jnp.int32)

    def step(_, lohi):
        lo, hi = lohi
        mid = lo + (hi - lo + 1) // 2
        c = jnp.sum((bits >= mid).astype(jnp.int32))
        ok = c >= K
        return jnp.where(ok, mid, lo), jnp.where(ok, hi, mid - 1)

    T, _ = lax.fori_loop(0, 31, step, (jnp.int32(0), jnp.int32(2**30)))

    maskG = (bits > T)
    nG = jnp.sum(maskG.astype(jnp.int32))
    maskE = (bits == T)
    r_fill = (K - nG).astype(jnp.float32)

    U = (lax.broadcasted_iota(jnp.int32, (R, R), 0)
         < lax.broadcasted_iota(jnp.int32, (R, R), 1)).astype(jnp.float32)
    Ls = U.T

    eM = maskE.astype(jnp.float32)
    prefE = jnp.dot(eM, U, preferred_element_type=jnp.float32)
    offE = jnp.dot(Ls, jnp.sum(eM, axis=1, keepdims=True),
                   preferred_element_type=jnp.float32)
    selE = jnp.logical_and(maskE, (offE + prefE) < r_fill)

    sM = jnp.logical_or(maskG, selE).astype(jnp.float32)
    prefS = jnp.dot(sM, U, preferred_element_type=jnp.float32)
    offS = jnp.dot(Ls, jnp.sum(sM, axis=1, keepdims=True),
                   preferred_element_type=jnp.float32)

    sIota = lax.broadcasted_iota(jnp.int32, (1, K), 1).astype(jnp.float32)
    rIota = lax.broadcasted_iota(jnp.int32, (1, R), 1).astype(jnp.float32)

    Mrs = (offS <= sIota).astype(jnp.float32)
    Rcol = (jnp.sum(Mrs, axis=0, keepdims=True) - 1.0).T
    offR = jnp.max(offS * Mrs, axis=0, keepdims=True)
    jcol = (sIota - offR).T

    OHT = (Rcol == rIota).astype(jnp.float32)
    prefRow = jnp.dot(OHT, prefS, preferred_element_type=jnp.float32)
    selRow = jnp.dot(OHT, sM, preferred_element_type=jnp.float32)
    aRow = jnp.dot(OHT, a, preferred_element_type=jnp.float32)

    OHc = ((prefRow == jcol) & (selRow > 0.5)).astype(jnp.float32)
    Ccol = jnp.sum(OHc * rIota, axis=1, keepdims=True)
    val = jnp.sum(OHc * aRow, axis=1, keepdims=True)
    idxf = Rcol * float(C_) + Ccol
    return idxf, val


def _k1_body(act_ref, idx_ref, val_ref, cnt_ref):
    a = act_ref[0]
    idxf, val = _select_topk(a)
    idx_ref[0] = idxf
    val_ref[0] = val
    cnt = jnp.sum((a > 0.01).astype(jnp.float32))
    cnt_ref[0] = jnp.zeros((K, 1), jnp.float32) + cnt


def _k5_body(sk_ref, act_ref, idx_ref, valm_ref, msk_ref):
    a = act_ref[0]
    idxf, val = _select_topk(a)
    vrow = val.T
    irow = idxf.T
    beats = ((vrow > val) | ((vrow == val) & (irow < idxf))).astype(jnp.float32)
    rank = jnp.sum(beats, axis=1, keepdims=True)
    skf = jnp.minimum(sk_ref[0], N).astype(jnp.float32)
    msk = (rank < skf).astype(jnp.float32)
    idx_ref[0] = idxf
    valm_ref[0] = val * msk
    msk_ref[0] = msk


def _k3_body(km_ref, x_ref, val_ref, idx_ref, wqkv_ref, bqkv_ref, wo_ref, bo_ref,
             msg_ref):
    x = x_ref[0]
    v = val_ref[0]
    idxf = idx_ref[0]
    kmf = km_ref[0] * jnp.float32(1.0)

    vrow = v.T
    irow = idxf.T
    beats = ((vrow > v) | ((vrow == v) & (irow < idxf))).astype(jnp.float32)
    rank = jnp.sum(beats, axis=1, keepdims=True)
    validc = (rank < kmf).astype(jnp.float32)
    valid_row = validc.T

    qkv = jnp.dot(x, wqkv_ref[...], preferred_element_type=jnp.float32) + bqkv_ref[...]
    dh = D // H
    scale = 1.0 / (dh ** 0.5)
    outs = []
    for h in range(H):
        q = qkv[:, h * dh:(h + 1) * dh]
        k = qkv[:, D + h * dh:D + (h + 1) * dh]
        vv = qkv[:, 2 * D + h * dh:2 * D + (h + 1) * dh]
        logits = jnp.dot(q, k.T, preferred_element_type=jnp.float32) * scale
        logits = jnp.where(valid_row > 0.5, logits, -1e30)
        m = jnp.max(logits, axis=1, keepdims=True)
        e = jnp.exp(logits - m)
        attn = e / jnp.sum(e, axis=1, keepdims=True)
        outs.append(jnp.dot(attn, vv, preferred_element_type=jnp.float32))
    o = jnp.concatenate(outs, axis=1)
    o = jnp.dot(o, wo_ref[...], preferred_element_type=jnp.float32) + bo_ref[...]
    msg_ref[0] = o * v * validc


def _k4_body(hid_ref, act_ref, idx_ref, msg_ref,
             w1a_ref, w1b_ref, b1_ref, w2_ref, b2_ref,
             a1a_ref, a1b_ref, ab1_ref, a2_ref, ab2_ref,
             lng_ref, lnb_ref, nh_ref, na_ref):
    c = pl.program_id(1)
    base = (c * CHUNK).astype(jnp.float32)
    hid = hid_ref[0]
    nIota = lax.broadcasted_iota(jnp.int32, (CHUNK, 1), 0).astype(jnp.float32)
    idxrow = idx_ref[0].T
    onehot = ((nIota + base) == idxrow).astype(jnp.float32)
    msgc = jnp.dot(onehot, msg_ref[0], preferred_element_type=jnp.float32)

    h = _gelu(jnp.dot(hid, w1a_ref[...], preferred_element_type=jnp.float32)
              + jnp.dot(msgc, w1b_ref[...], preferred_element_type=jnp.float32)
              + b1_ref[...])
    y = jnp.dot(h, w2_ref[...], preferred_element_type=jnp.float32) + b2_ref[...]
    mu = jnp.mean(y, axis=1, keepdims=True)
    yc = y - mu
    var = jnp.mean(yc * yc, axis=1, keepdims=True)
    nh = yc * lax.rsqrt(var + 1e-5) * lng_ref[...] + lnb_ref[...]

    a2 = _gelu(jnp.dot(hid, a1a_ref[...], preferred_element_type=jnp.float32)
               + jnp.dot(nh, a1b_ref[...], preferred_element_type=jnp.float32)
               + ab1_ref[...])
    dpre = jnp.dot(a2, a2_ref[...], preferred_element_type=jnp.float32) + ab2_ref[...]
    delta = jax.nn.sigmoid(dpre)
    na = jnp.clip(0.7 * act_ref[0] + 0.3 * delta, 0.0, 1.0)
    nh_ref[0] = nh
    na_ref[0] = na


def _k7_body(idx_ref, valm_ref, msk_ref, rows_ref, oh_ref, oa_ref):
    c = pl.program_id(1)
    base = (c * CHUNK).astype(jnp.float32)
    nIota = lax.broadcasted_iota(jnp.int32, (CHUNK, 1), 0).astype(jnp.float32)
    idxrow = idx_ref[0].T
    onehot = ((nIota + base) == idxrow).astype(jnp.float32)
    rows = rows_ref[0] * msk_ref[0]
    oh_ref[0] = jnp.dot(onehot, rows, preferred_element_type=jnp.float32)
    oa_ref[0] = jnp.dot(onehot, valm_ref[0], preferred_element_type=jnp.float32)


def _sc_gather(table, idx_flat):
    nidx = idx_flat.shape[0]
    per_w = nidx // NSC
    mesh = plsc.VectorSubcoreMesh(core_axis_name="c", subcore_axis_name="s")

    @functools.partial(
        pl.kernel, mesh=mesh,
        out_type=jax.ShapeDtypeStruct((nidx, D), jnp.float32),
        scratch_types=[
            pltpu.VMEM((per_w,), jnp.int32),
            pltpu.VMEM((per_w, D), jnp.float32),
            pltpu.SemaphoreType.DMA,
        ],
    )
    def gath(table_hbm, idx_hbm, out_hbm, idx_v, rows_v, sem):
        wid = lax.axis_index("s") * 2 + lax.axis_index("c")
        base = wid * per_w
        pltpu.sync_copy(idx_hbm.at[pl.ds(base, per_w)], idx_v)
        pltpu.async_copy(table_hbm.at[idx_v], rows_v, sem).wait()
        pltpu.sync_copy(rows_v, out_hbm.at[pl.ds(base, per_w)])

    return gath(table, idx_flat)


def kernel(activation, hidden_state, sparsity_k, in_proj_w, in_proj_b, out_w, out_b,
           su_w1, su_b1, su_w2, su_b2, au_w1, au_b1, au_w2, au_b2, ln_g, ln_b):
    f32 = jnp.float32
    act_sq = activation.reshape(B, R, C_)
    act_col = activation.reshape(B, N, 1)

    idx1, val1, cnt = pl.pallas_call(
        _k1_body,
        grid=(B,),
        in_specs=[pl.BlockSpec((1, R, C_), lambda b: (b, 0, 0))],
        out_specs=[pl.BlockSpec((1, K, 1), lambda b: (b, 0, 0))] * 3,
        out_shape=[jax.ShapeDtypeStruct((B, K, 1), f32)] * 3,
    )(act_sq)

    k_msg = jnp.maximum(jnp.minimum(jnp.max(cnt[:, 0, 0]), 256.0), 1.0)
    k_msg_arr = k_msg.astype(jnp.int32).reshape(1)

    gidx1 = (idx1[:, :, 0].astype(jnp.int32)
             + jnp.arange(B, dtype=jnp.int32)[:, None] * N).reshape(B * K)
    active = _sc_gather(hidden_state.reshape(B * N, D), gidx1).reshape(B, K, D)

    wqkv_t = in_proj_w.T
    bqkv_r = in_proj_b.reshape(1, 3 * D)
    wo_t = out_w.T
    bo_r = out_b.reshape(1, D)
    msg = pl.pallas_call(
        _k3_body,
        grid=(B,),
        in_specs=[
            pl.BlockSpec(memory_space=pltpu.SMEM),
            pl.BlockSpec((1, K, D), lambda b: (b, 0, 0)),
            pl.BlockSpec((1, K, 1), lambda b: (b, 0, 0)),
            pl.BlockSpec((1, K, 1), lambda b: (b, 0, 0)),
            pl.BlockSpec((D, 3 * D), lambda b: (0, 0)),
            pl.BlockSpec((1, 3 * D), lambda b: (0, 0)),
            pl.BlockSpec((D, D), lambda b: (0, 0)),
            pl.BlockSpec((1, D), lambda b: (0, 0)),
        ],
        out_specs=pl.BlockSpec((1, K, D), lambda b: (b, 0, 0)),
        out_shape=jax.ShapeDtypeStruct((B, K, D), f32),
    )(k_msg_arr, active, val1, idx1, wqkv_t, bqkv_r, wo_t, bo_r)

    w1a = su_w1[:, :D].T
    w1b = su_w1[:, D:].T
    b1 = su_b1.reshape(1, D)
    w2 = su_w2.T
    b2 = su_b2.reshape(1, D)
    a1a = au_w1[:, :D].T
    a1b = au_w1[:, D:].T
    ab1 = au_b1.reshape(1, D)
    a2w = au_w2.T
    ab2 = au_b2.reshape(1, 1)
    lng = ln_g.reshape(1, D)
    lnb = ln_b.reshape(1, D)

    nchunks = N // CHUNK
    new_hidden, new_act = pl.pallas_call(
        _k4_body,
        grid=(B, nchunks),
        in_specs=[
            pl.BlockSpec((1, CHUNK, D), lambda b, c: (b, c, 0)),
            pl.BlockSpec((1, CHUNK, 1), lambda b, c: (b, c, 0)),
            pl.BlockSpec((1, K, 1), lambda b, c: (b, 0, 0)),
            pl.BlockSpec((1, K, D), lambda b, c: (b, 0, 0)),
            pl.BlockSpec((D, D), lambda b, c: (0, 0)),
            pl.BlockSpec((D, D), lambda b, c: (0, 0)),
            pl.BlockSpec((1, D), lambda b, c: (0, 0)),
            pl.BlockSpec((D, D), lambda b, c: (0, 0)),
            pl.BlockSpec((1, D), lambda b, c: (0, 0)),
            pl.BlockSpec((D, D), lambda b, c: (0, 0)),
            pl.BlockSpec((D, D), lambda b, c: (0, 0)),
            pl.BlockSpec((1, D), lambda b, c: (0, 0)),
            pl.BlockSpec((D, 1), lambda b, c: (0, 0)),
            pl.BlockSpec((1, 1), lambda b, c: (0, 0)),
            pl.BlockSpec((1, D), lambda b, c: (0, 0)),
            pl.BlockSpec((1, D), lambda b, c: (0, 0)),
        ],
        out_specs=[
            pl.BlockSpec((1, CHUNK, D), lambda b, c: (b, c, 0)),
            pl.BlockSpec((1, CHUNK, 1), lambda b, c: (b, c, 0)),
        ],
        out_shape=[
            jax.ShapeDtypeStruct((B, N, D), f32),
            jax.ShapeDtypeStruct((B, N, 1), f32),
        ],
    )(hidden_state, act_col, idx1, msg,
      w1a, w1b, b1, w2, b2, a1a, a1b, ab1, a2w, ab2, lng, lnb)

    sk_arr = jnp.asarray(sparsity_k, jnp.int32).reshape(1)
    idx2, valm, msk = pl.pallas_call(
        _k5_body,
        grid=(B,),
        in_specs=[
            pl.BlockSpec(memory_space=pltpu.SMEM),
            pl.BlockSpec((1, R, C_), lambda b: (b, 0, 0)),
        ],
        out_specs=[pl.BlockSpec((1, K, 1), lambda b: (b, 0, 0))] * 3,
        out_shape=[jax.ShapeDtypeStruct((B, K, 1), f32)] * 3,
    )(sk_arr, new_act.reshape(B, R, C_))

    gidx2 = (idx2[:, :, 0].astype(jnp.int32)
             + jnp.arange(B, dtype=jnp.int32)[:, None] * N).reshape(B * K)
    sel_rows = _sc_gather(new_hidden.reshape(B * N, D), gidx2).reshape(B, K, D)

    out_hid, out_act = pl.pallas_call(
        _k7_body,
        grid=(B, nchunks),
        in_specs=[
            pl.BlockSpec((1, K, 1), lambda b, c: (b, 0, 0)),
            pl.BlockSpec((1, K, 1), lambda b, c: (b, 0, 0)),
            pl.BlockSpec((1, K, 1), lambda b, c: (b, 0, 0)),
            pl.BlockSpec((1, K, D), lambda b, c: (b, 0, 0)),
        ],
        out_specs=[
            pl.BlockSpec((1, CHUNK, D), lambda b, c: (b, c, 0)),
            pl.BlockSpec((1, CHUNK, 1), lambda b, c: (b, c, 0)),
        ],
        out_shape=[
            jax.ShapeDtypeStruct((B, N, D), f32),
            jax.ShapeDtypeStruct((B, N, 1), f32),
        ],
    )(idx2, valm, msk, sel_rows)

    return out_act.reshape(B, N), out_hid


# speedup vs baseline: 2.6422x; 1.3222x over previous
import functools

import jax
import jax.numpy as jnp
from jax import lax
from jax.experimental import pallas as pl
from jax.experimental.pallas import tpu as pltpu
from jax.experimental.pallas import tpu_sc as plsc

B, N, D, H = 16, 16384, 128, 4
K = 256
R = 128
C_ = 128
CHUNK = 1024
NSC = 32


def _gelu(x):
    return 0.5 * x * (1.0 + lax.erf(x * 0.7071067811865476))


def _select_topk(a):
    bits = lax.bitcast_convert_type(a, jnp.int32)

    def step(_, lohi):
        lo, hi = lohi
        mid = lo + (hi - lo + 1) // 2
        c = jnp.sum((bits >= mid).astype(jnp.int32))
        ok = c >= K
        return jnp.where(ok, mid, lo), jnp.where(ok, hi, mid - 1)

    T, _ = lax.fori_loop(0, 31, step, (jnp.int32(0), jnp.int32(2**30)))

    maskG = (bits > T)
    nG = jnp.sum(maskG.astype(jnp.int32))
    maskE = (bits == T)
    r_fill = (K - nG).astype(jnp.float32)

    U = (lax.broadcasted_iota(jnp.int32, (R, R), 0)
         < lax.broadcasted_iota(jnp.int32, (R, R), 1)).astype(jnp.float32)
    Ls = U.T

    eM = maskE.astype(jnp.float32)
    prefE = jnp.dot(eM, U, preferred_element_type=jnp.float32)
    offE = jnp.dot(Ls, jnp.sum(eM, axis=1, keepdims=True),
                   preferred_element_type=jnp.float32)
    selE = jnp.logical_and(maskE, (offE + prefE) < r_fill)

    sM = jnp.logical_or(maskG, selE).astype(jnp.float32)
    prefS = jnp.dot(sM, U, preferred_element_type=jnp.float32)
    offS = jnp.dot(Ls, jnp.sum(sM, axis=1, keepdims=True),
                   preferred_element_type=jnp.float32)

    sIota = lax.broadcasted_iota(jnp.int32, (1, K), 1).astype(jnp.float32)
    rIota = lax.broadcasted_iota(jnp.int32, (1, R), 1).astype(jnp.float32)

    Mrs = (offS <= sIota).astype(jnp.float32)
    Rcol = (jnp.sum(Mrs, axis=0, keepdims=True) - 1.0).T
    offR = jnp.max(offS * Mrs, axis=0, keepdims=True)
    jcol = (sIota - offR).T

    OHT = (Rcol == rIota).astype(jnp.float32)
    prefRow = jnp.dot(OHT, prefS, preferred_element_type=jnp.float32)
    selRow = jnp.dot(OHT, sM, preferred_element_type=jnp.float32)
    aRow = jnp.dot(OHT, a, preferred_element_type=jnp.float32)

    OHc = ((prefRow == jcol) & (selRow > 0.5)).astype(jnp.float32)
    Ccol = jnp.sum(OHc * rIota, axis=1, keepdims=True)
    val = jnp.sum(OHc * aRow, axis=1, keepdims=True)
    idxf = Rcol * float(C_) + Ccol
    return idxf, val


def _k1_body(act_ref, idx_ref, val_ref, cnt_ref):
    a = act_ref[0]
    idxf, val = _select_topk(a)
    idx_ref[0] = idxf
    val_ref[0] = val
    cnt = jnp.sum((a > 0.01).astype(jnp.float32))
    cnt_ref[0] = jnp.zeros((K, 1), jnp.float32) + cnt


def _k5_body(sk_ref, act_ref, idx_ref, valm_ref, msk_ref):
    a = act_ref[0]
    idxf, val = _select_topk(a)
    vrow = val.T
    irow = idxf.T
    beats = ((vrow > val) | ((vrow == val) & (irow < idxf))).astype(jnp.float32)
    rank = jnp.sum(beats, axis=1, keepdims=True)
    skf = jnp.minimum(sk_ref[0], N).astype(jnp.float32)
    msk = (rank < skf).astype(jnp.float32)
    idx_ref[0] = idxf
    valm_ref[0] = val * msk
    msk_ref[0] = msk


def _k3_body(km_ref, x_ref, val_ref, idx_ref, wqkv_ref, bqkv_ref, wo_ref, bo_ref,
             msg_ref):
    x = x_ref[0]
    v = val_ref[0]
    idxf = idx_ref[0]
    kmf = km_ref[0] * jnp.float32(1.0)

    vrow = v.T
    irow = idxf.T
    beats = ((vrow > v) | ((vrow == v) & (irow < idxf))).astype(jnp.float32)
    rank = jnp.sum(beats, axis=1, keepdims=True)
    validc = (rank < kmf).astype(jnp.float32)
    valid_row = validc.T

    qkv = jnp.dot(x, wqkv_ref[...], preferred_element_type=jnp.float32) + bqkv_ref[...]
    dh = D // H
    scale = 1.0 / (dh ** 0.5)
    outs = []
    for h in range(H):
        q = qkv[:, h * dh:(h + 1) * dh]
        k = qkv[:, D + h * dh:D + (h + 1) * dh]
        vv = qkv[:, 2 * D + h * dh:2 * D + (h + 1) * dh]
        logits = jnp.dot(q, k.T, preferred_element_type=jnp.float32) * scale
        logits = jnp.where(valid_row > 0.5, logits, -1e30)
        m = jnp.max(logits, axis=1, keepdims=True)
        e = jnp.exp(logits - m)
        attn = e / jnp.sum(e, axis=1, keepdims=True)
        outs.append(jnp.dot(attn, vv, preferred_element_type=jnp.float32))
    o = jnp.concatenate(outs, axis=1)
    o = jnp.dot(o, wo_ref[...], preferred_element_type=jnp.float32) + bo_ref[...]
    msg_ref[0] = o * v * validc


def _k4_body(hid_ref, act_ref, idx_ref, msg_ref,
             w1a_ref, w1b_ref, b1_ref, w2_ref, b2_ref,
             a1a_ref, a1b_ref, ab1_ref, a2_ref, ab2_ref,
             lng_ref, lnb_ref, nh_ref, na_ref):
    c = pl.program_id(1)
    base = (c * CHUNK).astype(jnp.float32)
    hid = hid_ref[0]
    nIota = lax.broadcasted_iota(jnp.int32, (CHUNK, 1), 0).astype(jnp.float32)
    idxrow = idx_ref[0].T
    onehot = ((nIota + base) == idxrow).astype(jnp.float32)
    msgc = jnp.dot(onehot, msg_ref[0], preferred_element_type=jnp.float32)

    h = _gelu(jnp.dot(hid, w1a_ref[...], preferred_element_type=jnp.float32)
              + jnp.dot(msgc, w1b_ref[...], preferred_element_type=jnp.float32)
              + b1_ref[...])
    y = jnp.dot(h, w2_ref[...], preferred_element_type=jnp.float32) + b2_ref[...]
    mu = jnp.mean(y, axis=1, keepdims=True)
    yc = y - mu
    var = jnp.mean(yc * yc, axis=1, keepdims=True)
    nh = yc * lax.rsqrt(var + 1e-5) * lng_ref[...] + lnb_ref[...]

    a2 = _gelu(jnp.dot(hid, a1a_ref[...], preferred_element_type=jnp.float32)
               + jnp.dot(nh, a1b_ref[...], preferred_element_type=jnp.float32)
               + ab1_ref[...])
    dpre = jnp.dot(a2, a2_ref[...], preferred_element_type=jnp.float32) + ab2_ref[...]
    delta = jax.nn.sigmoid(dpre)
    na = jnp.clip(0.7 * act_ref[0] + 0.3 * delta, 0.0, 1.0)
    nh_ref[0] = nh
    na_ref[0] = na


def _k7_body(idx_ref, valm_ref, msk_ref, rows_ref, oh_ref, oa_ref):
    c = pl.program_id(1)
    base = (c * CHUNK).astype(jnp.float32)
    nIota = lax.broadcasted_iota(jnp.int32, (CHUNK, 1), 0).astype(jnp.float32)
    idxrow = idx_ref[0].T
    onehot = ((nIota + base) == idxrow).astype(jnp.float32)
    rows = rows_ref[0] * msk_ref[0]
    oh_ref[0] = jnp.dot(onehot, rows, preferred_element_type=jnp.float32)
    oa_ref[0] = jnp.dot(onehot, valm_ref[0], preferred_element_type=jnp.float32)


def _sc_gather(table, idx_flat):
    nidx = idx_flat.shape[0]
    per_w = nidx // NSC
    mesh = plsc.VectorSubcoreMesh(core_axis_name="c", subcore_axis_name="s")

    @functools.partial(
        pl.kernel, mesh=mesh,
        out_type=jax.ShapeDtypeStruct((nidx, D), jnp.float32),
        scratch_types=[
            pltpu.VMEM((per_w,), jnp.int32),
            pltpu.VMEM((per_w, D), jnp.float32),
            pltpu.SemaphoreType.DMA,
        ],
    )
    def gath(table_hbm, idx_hbm, out_hbm, idx_v, rows_v, sem):
        wid = lax.axis_index("s") * 2 + lax.axis_index("c")
        base = wid * per_w
        pltpu.sync_copy(idx_hbm.at[pl.ds(base, per_w)], idx_v)
        pltpu.async_copy(table_hbm.at[idx_v], rows_v, sem).wait()
        pltpu.sync_copy(rows_v, out_hbm.at[pl.ds(base, per_w)])

    return gath(table, idx_flat)


def kernel(activation, hidden_state, sparsity_k, in_proj_w, in_proj_b, out_w, out_b,
           su_w1, su_b1, su_w2, su_b2, au_w1, au_b1, au_w2, au_b2, ln_g, ln_b):
    f32 = jnp.float32
    act_sq = activation.reshape(B, R, C_)
    act_col = activation.reshape(B, N, 1)

    idx1, val1, cnt = pl.pallas_call(
        _k1_body,
        grid=(B,),
        in_specs=[pl.BlockSpec((1, R, C_), lambda b: (b, 0, 0))],
        out_specs=[pl.BlockSpec((1, K, 1), lambda b: (b, 0, 0))] * 3,
        out_shape=[jax.ShapeDtypeStruct((B, K, 1), f32)] * 3,
    )(act_sq)

    k_msg = jnp.maximum(jnp.minimum(jnp.max(cnt[:, 0, 0]), 256.0), 1.0)
    k_msg_arr = k_msg.astype(jnp.int32).reshape(1)

    gidx1 = (idx1[:, :, 0].astype(jnp.int32)
             + jnp.arange(B, dtype=jnp.int32)[:, None] * N).reshape(B * K)
    active = _sc_gather(hidden_state.reshape(B * N, D), gidx1).reshape(B, K, D)

    wqkv_t = in_proj_w.T
    bqkv_r = in_proj_b.reshape(1, 3 * D)
    wo_t = out_w.T
    bo_r = out_b.reshape(1, D)
    msg = pl.pallas_call(
        _k3_body,
        grid=(B,),
        in_specs=[
            pl.BlockSpec(memory_space=pltpu.SMEM),
            pl.BlockSpec((1, K, D), lambda b: (b, 0, 0)),
            pl.BlockSpec((1, K, 1), lambda b: (b, 0, 0)),
            pl.BlockSpec((1, K, 1), lambda b: (b, 0, 0)),
            pl.BlockSpec((D, 3 * D), lambda b: (0, 0)),
            pl.BlockSpec((1, 3 * D), lambda b: (0, 0)),
            pl.BlockSpec((D, D), lambda b: (0, 0)),
            pl.BlockSpec((1, D), lambda b: (0, 0)),
        ],
        out_specs=pl.BlockSpec((1, K, D), lambda b: (b, 0, 0)),
        out_shape=jax.ShapeDtypeStruct((B, K, D), f32),
    )(k_msg_arr, active, val1, idx1, wqkv_t, bqkv_r, wo_t, bo_r)

    w1a = su_w1[:, :D].T
    w1b = su_w1[:, D:].T
    b1 = su_b1.reshape(1, D)
    w2 = su_w2.T
    b2 = su_b2.reshape(1, D)
    a1a = au_w1[:, :D].T
    a1b = au_w1[:, D:].T
    ab1 = au_b1.reshape(1, D)
    a2w = au_w2.T
    ab2 = au_b2.reshape(1, 1)
    lng = ln_g.reshape(1, D)
    lnb = ln_b.reshape(1, D)

    nchunks = N // CHUNK
    new_hidden, new_act = pl.pallas_call(
        _k4_body,
        grid=(B, nchunks),
        in_specs=[
            pl.BlockSpec((1, CHUNK, D), lambda b, c: (b, c, 0)),
            pl.BlockSpec((1, CHUNK, 1), lambda b, c: (b, c, 0)),
            pl.BlockSpec((1, K, 1), lambda b, c: (b, 0, 0)),
            pl.BlockSpec((1, K, D), lambda b, c: (b, 0, 0)),
            pl.BlockSpec((D, D), lambda b, c: (0, 0)),
            pl.BlockSpec((D, D), lambda b, c: (0, 0)),
            pl.BlockSpec((1, D), lambda b, c: (0, 0)),
            pl.BlockSpec((D, D), lambda b, c: (0, 0)),
            pl.BlockSpec((1, D), lambda b, c: (0, 0)),
            pl.BlockSpec((D, D), lambda b, c: (0, 0)),
            pl.BlockSpec((D, D), lambda b, c: (0, 0)),
            pl.BlockSpec((1, D), lambda b, c: (0, 0)),
            pl.BlockSpec((D, 1), lambda b, c: (0, 0)),
            pl.BlockSpec((1, 1), lambda b, c: (0, 0)),
            pl.BlockSpec((1, D), lambda b, c: (0, 0)),
            pl.BlockSpec((1, D), lambda b, c: (0, 0)),
        ],
        out_specs=[
            pl.BlockSpec((1, CHUNK, D), lambda b, c: (b, c, 0)),
            pl.BlockSpec((1, CHUNK, 1), lambda b, c: (b, c, 0)),
        ],
        out_shape=[
            jax.ShapeDtypeStruct((B, N, D), f32),
            jax.ShapeDtypeStruct((B, N, 1), f32),
        ],
    )(hidden_state, act_col, idx1, msg,
      w1a, w1b, b1, w2, b2, a1a, a1b, ab1, a2w, ab2, lng, lnb)

    sk_arr = jnp.asarray(sparsity_k, jnp.int32).reshape(1)
    idx2, valm, msk = pl.pallas_call(
        _k5_body,
        grid=(B,),
        in_specs=[
            pl.BlockSpec(memory_space=pltpu.SMEM),
            pl.BlockSpec((1, R, C_), lambda b: (b, 0, 0)),
        ],
        out_specs=[pl.BlockSpec((1, K, 1), lambda b: (b, 0, 0))] * 3,
        out_shape=[jax.ShapeDtypeStruct((B, K, 1), f32)] * 3,
    )(sk_arr, new_act.reshape(B, R, C_))

    gidx2 = (idx2[:, :, 0].astype(jnp.int32)
             + jnp.arange(B, dtype=jnp.int32)[:, None] * N).reshape(B * K)
    sel_rows = _sc_gather(new_hidden.reshape(B * N, D), gidx2).reshape(B, K, D)

    out_hid, out_act = pl.pallas_call(
        _k7_body,
        grid=(B, nchunks),
        in_specs=[
            pl.BlockSpec((1, K, 1), lambda b, c: (b, 0, 0)),
            pl.BlockSpec((1, K, 1), lambda b, c: (b, 0, 0)),
            pl.BlockSpec((1, K, 1), lambda b, c: (b, 0, 0)),
            pl.BlockSpec((1, K, D), lambda b, c: (b, 0, 0)),
        ],
        out_specs=[
            pl.BlockSpec((1, CHUNK, D), lambda b, c: (b, c, 0)),
            pl.BlockSpec((1, CHUNK, 1), lambda b, c: (b, c, 0)),
        ],
        out_shape=[
            jax.ShapeDtypeStruct((B, N, D), f32),
            jax.ShapeDtypeStruct((B, N, 1), f32),
        ],
    )(idx2, valm, msk, sel_rows)

    return out_act.reshape(B, N), out_hid


# speedup vs baseline: 3.0834x; 1.1670x over previous
import functools

import jax
import jax.numpy as jnp
from jax import lax
from jax.experimental import pallas as pl
from jax.experimental.pallas import tpu as pltpu
from jax.experimental.pallas import tpu_sc as plsc

B, N, D, H = 16, 16384, 128, 4
K = 256
R = 128
C_ = 128
CHUNK = 2048
NSC = 32


def _gelu(x):
    return 0.5 * x * (1.0 + lax.erf(x * 0.7071067811865476))


def _select_topk(a):
    bits = lax.bitcast_convert_type(a, jnp.int32)

    def step(_, lohi):
        lo, hi = lohi
        mid = lo + (hi - lo + 1) // 2
        c = jnp.sum((bits >= mid).astype(jnp.int32))
        ok = c >= K
        return jnp.where(ok, mid, lo), jnp.where(ok, hi, mid - 1)

    T, _ = lax.fori_loop(0, 31, step, (jnp.int32(0), jnp.int32(2**30)))

    maskG = (bits > T)
    nG = jnp.sum(maskG.astype(jnp.int32))
    maskE = (bits == T)
    r_fill = (K - nG).astype(jnp.float32)

    U = (lax.broadcasted_iota(jnp.int32, (R, R), 0)
         < lax.broadcasted_iota(jnp.int32, (R, R), 1)).astype(jnp.float32)
    Ls = U.T

    eM = maskE.astype(jnp.float32)
    prefE = jnp.dot(eM, U, preferred_element_type=jnp.float32)
    offE = jnp.dot(Ls, jnp.sum(eM, axis=1, keepdims=True),
                   preferred_element_type=jnp.float32)
    selE = jnp.logical_and(maskE, (offE + prefE) < r_fill)

    sM = jnp.logical_or(maskG, selE).astype(jnp.float32)
    prefS = jnp.dot(sM, U, preferred_element_type=jnp.float32)
    offS = jnp.dot(Ls, jnp.sum(sM, axis=1, keepdims=True),
                   preferred_element_type=jnp.float32)

    sIota = lax.broadcasted_iota(jnp.int32, (1, K), 1).astype(jnp.float32)
    rIota = lax.broadcasted_iota(jnp.int32, (1, R), 1).astype(jnp.float32)

    Mrs = (offS <= sIota).astype(jnp.float32)
    Rcol = (jnp.sum(Mrs, axis=0, keepdims=True) - 1.0).T
    offR = jnp.max(offS * Mrs, axis=0, keepdims=True)
    jcol = (sIota - offR).T

    OHT = (Rcol == rIota).astype(jnp.float32)
    prefRow = jnp.dot(OHT, prefS, preferred_element_type=jnp.float32)
    selRow = jnp.dot(OHT, sM, preferred_element_type=jnp.float32)
    aRow = jnp.dot(OHT, a, preferred_element_type=jnp.float32)

    OHc = ((prefRow == jcol) & (selRow > 0.5)).astype(jnp.float32)
    Ccol = jnp.sum(OHc * rIota, axis=1, keepdims=True)
    val = jnp.sum(OHc * aRow, axis=1, keepdims=True)
    idxf = Rcol * float(C_) + Ccol
    return idxf, val


def _k1_body(act_ref, idx_ref, val_ref, cnt_ref):
    a = act_ref[0]
    idxf, val = _select_topk(a)
    idx_ref[0] = idxf
    val_ref[0] = val
    cnt = jnp.sum((a > 0.01).astype(jnp.float32))
    cnt_ref[0] = jnp.zeros((K, 1), jnp.float32) + cnt


def _k5_body(sk_ref, act_ref, idx_ref, valm_ref, msk_ref):
    a = act_ref[0]
    idxf, val = _select_topk(a)
    vrow = val.T
    irow = idxf.T
    beats = ((vrow > val) | ((vrow == val) & (irow < idxf))).astype(jnp.float32)
    rank = jnp.sum(beats, axis=1, keepdims=True)
    skf = jnp.minimum(sk_ref[0], N).astype(jnp.float32)
    msk = (rank < skf).astype(jnp.float32)
    idx_ref[0] = idxf
    valm_ref[0] = val * msk
    msk_ref[0] = msk


def _k3_body(km_ref, x_ref, val_ref, idx_ref, wqkv_ref, bqkv_ref, wo_ref, bo_ref,
             msg_ref):
    x = x_ref[0]
    v = val_ref[0]
    idxf = idx_ref[0]
    kmf = km_ref[0] * jnp.float32(1.0)

    vrow = v.T
    irow = idxf.T
    beats = ((vrow > v) | ((vrow == v) & (irow < idxf))).astype(jnp.float32)
    rank = jnp.sum(beats, axis=1, keepdims=True)
    validc = (rank < kmf).astype(jnp.float32)
    valid_row = validc.T

    qkv = jnp.dot(x, wqkv_ref[...], preferred_element_type=jnp.float32) + bqkv_ref[...]
    dh = D // H
    scale = 1.0 / (dh ** 0.5)
    outs = []
    for h in range(H):
        q = qkv[:, h * dh:(h + 1) * dh]
        k = qkv[:, D + h * dh:D + (h + 1) * dh]
        vv = qkv[:, 2 * D + h * dh:2 * D + (h + 1) * dh]
        logits = jnp.dot(q, k.T, preferred_element_type=jnp.float32) * scale
        logits = jnp.where(valid_row > 0.5, logits, -1e30)
        m = jnp.max(logits, axis=1, keepdims=True)
        e = jnp.exp(logits - m)
        attn = e / jnp.sum(e, axis=1, keepdims=True)
        outs.append(jnp.dot(attn, vv, preferred_element_type=jnp.float32))
    o = jnp.concatenate(outs, axis=1)
    o = jnp.dot(o, wo_ref[...], preferred_element_type=jnp.float32) + bo_ref[...]
    msg_ref[0] = o * v * validc


def _k4_body(hid_ref, act_ref, idx_ref, msg_ref,
             w1a_ref, w1b_ref, b1_ref, w2_ref, b2_ref,
             a1a_ref, a1b_ref, ab1_ref, a2_ref, ab2_ref,
             lng_ref, lnb_ref, nh_ref, na_ref):
    c = pl.program_id(1)
    base = (c * CHUNK).astype(jnp.float32)
    hid = hid_ref[0]
    nIota = lax.broadcasted_iota(jnp.int32, (CHUNK, 1), 0).astype(jnp.float32)
    idxrow = idx_ref[0].T
    onehot = ((nIota + base) == idxrow).astype(jnp.float32)
    msgc = jnp.dot(onehot, msg_ref[0], preferred_element_type=jnp.float32)

    h = _gelu(jnp.dot(hid, w1a_ref[...], preferred_element_type=jnp.float32)
              + jnp.dot(msgc, w1b_ref[...], preferred_element_type=jnp.float32)
              + b1_ref[...])
    y = jnp.dot(h, w2_ref[...], preferred_element_type=jnp.float32) + b2_ref[...]
    mu = jnp.mean(y, axis=1, keepdims=True)
    yc = y - mu
    var = jnp.mean(yc * yc, axis=1, keepdims=True)
    nh = yc * lax.rsqrt(var + 1e-5) * lng_ref[...] + lnb_ref[...]

    a2 = _gelu(jnp.dot(hid, a1a_ref[...], preferred_element_type=jnp.float32)
               + jnp.dot(nh, a1b_ref[...], preferred_element_type=jnp.float32)
               + ab1_ref[...])
    dpre = jnp.dot(a2, a2_ref[...], preferred_element_type=jnp.float32) + ab2_ref[...]
    delta = jax.nn.sigmoid(dpre)
    na = jnp.clip(0.7 * act_ref[0] + 0.3 * delta, 0.0, 1.0)
    nh_ref[0] = nh
    na_ref[0] = na


def _k7_body(idx_ref, valm_ref, msk_ref, rows_ref, oh_ref, oa_ref):
    c = pl.program_id(1)
    base = (c * CHUNK).astype(jnp.float32)
    nIota = lax.broadcasted_iota(jnp.int32, (CHUNK, 1), 0).astype(jnp.float32)
    idxrow = idx_ref[0].T
    onehot = ((nIota + base) == idxrow).astype(jnp.float32)
    rows = rows_ref[0] * msk_ref[0]
    oh_ref[0] = jnp.dot(onehot, rows, preferred_element_type=jnp.float32)
    oa_ref[0] = jnp.dot(onehot, valm_ref[0], preferred_element_type=jnp.float32)


def _sc_gather(table, idx_flat):
    nidx = idx_flat.shape[0]
    per_w = nidx // NSC
    mesh = plsc.VectorSubcoreMesh(core_axis_name="c", subcore_axis_name="s")

    @functools.partial(
        pl.kernel, mesh=mesh,
        out_type=jax.ShapeDtypeStruct((nidx, D), jnp.float32),
        scratch_types=[
            pltpu.VMEM((per_w,), jnp.int32),
            pltpu.VMEM((per_w, D), jnp.float32),
            pltpu.SemaphoreType.DMA,
        ],
    )
    def gath(table_hbm, idx_hbm, out_hbm, idx_v, rows_v, sem):
        wid = lax.axis_index("s") * 2 + lax.axis_index("c")
        base = wid * per_w
        pltpu.sync_copy(idx_hbm.at[pl.ds(base, per_w)], idx_v)
        pltpu.async_copy(table_hbm.at[idx_v], rows_v, sem).wait()
        pltpu.sync_copy(rows_v, out_hbm.at[pl.ds(base, per_w)])

    return gath(table, idx_flat)


def kernel(activation, hidden_state, sparsity_k, in_proj_w, in_proj_b, out_w, out_b,
           su_w1, su_b1, su_w2, su_b2, au_w1, au_b1, au_w2, au_b2, ln_g, ln_b):
    f32 = jnp.float32
    act_sq = activation.reshape(B, R, C_)
    act_col = activation.reshape(B, N, 1)

    idx1, val1, cnt = pl.pallas_call(
        _k1_body,
        grid=(B,),
        in_specs=[pl.BlockSpec((1, R, C_), lambda b: (b, 0, 0))],
        out_specs=[pl.BlockSpec((1, K, 1), lambda b: (b, 0, 0))] * 3,
        out_shape=[jax.ShapeDtypeStruct((B, K, 1), f32)] * 3,
    )(act_sq)

    k_msg = jnp.maximum(jnp.minimum(jnp.max(cnt[:, 0, 0]), 256.0), 1.0)
    k_msg_arr = k_msg.astype(jnp.int32).reshape(1)

    gidx1 = (idx1[:, :, 0].astype(jnp.int32)
             + jnp.arange(B, dtype=jnp.int32)[:, None] * N).reshape(B * K)
    active = _sc_gather(hidden_state.reshape(B * N, D), gidx1).reshape(B, K, D)

    wqkv_t = in_proj_w.T
    bqkv_r = in_proj_b.reshape(1, 3 * D)
    wo_t = out_w.T
    bo_r = out_b.reshape(1, D)
    msg = pl.pallas_call(
        _k3_body,
        grid=(B,),
        in_specs=[
            pl.BlockSpec(memory_space=pltpu.SMEM),
            pl.BlockSpec((1, K, D), lambda b: (b, 0, 0)),
            pl.BlockSpec((1, K, 1), lambda b: (b, 0, 0)),
            pl.BlockSpec((1, K, 1), lambda b: (b, 0, 0)),
            pl.BlockSpec((D, 3 * D), lambda b: (0, 0)),
            pl.BlockSpec((1, 3 * D), lambda b: (0, 0)),
            pl.BlockSpec((D, D), lambda b: (0, 0)),
            pl.BlockSpec((1, D), lambda b: (0, 0)),
        ],
        out_specs=pl.BlockSpec((1, K, D), lambda b: (b, 0, 0)),
        out_shape=jax.ShapeDtypeStruct((B, K, D), f32),
    )(k_msg_arr, active, val1, idx1, wqkv_t, bqkv_r, wo_t, bo_r)

    w1a = su_w1[:, :D].T
    w1b = su_w1[:, D:].T
    b1 = su_b1.reshape(1, D)
    w2 = su_w2.T
    b2 = su_b2.reshape(1, D)
    a1a = au_w1[:, :D].T
    a1b = au_w1[:, D:].T
    ab1 = au_b1.reshape(1, D)
    a2w = au_w2.T
    ab2 = au_b2.reshape(1, 1)
    lng = ln_g.reshape(1, D)
    lnb = ln_b.reshape(1, D)

    nchunks = N // CHUNK
    new_hidden, new_act = pl.pallas_call(
        _k4_body,
        grid=(B, nchunks),
        in_specs=[
            pl.BlockSpec((1, CHUNK, D), lambda b, c: (b, c, 0)),
            pl.BlockSpec((1, CHUNK, 1), lambda b, c: (b, c, 0)),
            pl.BlockSpec((1, K, 1), lambda b, c: (b, 0, 0)),
            pl.BlockSpec((1, K, D), lambda b, c: (b, 0, 0)),
            pl.BlockSpec((D, D), lambda b, c: (0, 0)),
            pl.BlockSpec((D, D), lambda b, c: (0, 0)),
            pl.BlockSpec((1, D), lambda b, c: (0, 0)),
            pl.BlockSpec((D, D), lambda b, c: (0, 0)),
            pl.BlockSpec((1, D), lambda b, c: (0, 0)),
            pl.BlockSpec((D, D), lambda b, c: (0, 0)),
            pl.BlockSpec((D, D), lambda b, c: (0, 0)),
            pl.BlockSpec((1, D), lambda b, c: (0, 0)),
            pl.BlockSpec((D, 1), lambda b, c: (0, 0)),
            pl.BlockSpec((1, 1), lambda b, c: (0, 0)),
            pl.BlockSpec((1, D), lambda b, c: (0, 0)),
            pl.BlockSpec((1, D), lambda b, c: (0, 0)),
        ],
        out_specs=[
            pl.BlockSpec((1, CHUNK, D), lambda b, c: (b, c, 0)),
            pl.BlockSpec((1, CHUNK, 1), lambda b, c: (b, c, 0)),
        ],
        out_shape=[
            jax.ShapeDtypeStruct((B, N, D), f32),
            jax.ShapeDtypeStruct((B, N, 1), f32),
        ],
    )(hidden_state, act_col, idx1, msg,
      w1a, w1b, b1, w2, b2, a1a, a1b, ab1, a2w, ab2, lng, lnb)

    sk_arr = jnp.asarray(sparsity_k, jnp.int32).reshape(1)
    idx2, valm, msk = pl.pallas_call(
        _k5_body,
        grid=(B,),
        in_specs=[
            pl.BlockSpec(memory_space=pltpu.SMEM),
            pl.BlockSpec((1, R, C_), lambda b: (b, 0, 0)),
        ],
        out_specs=[pl.BlockSpec((1, K, 1), lambda b: (b, 0, 0))] * 3,
        out_shape=[jax.ShapeDtypeStruct((B, K, 1), f32)] * 3,
    )(sk_arr, new_act.reshape(B, R, C_))

    gidx2 = (idx2[:, :, 0].astype(jnp.int32)
             + jnp.arange(B, dtype=jnp.int32)[:, None] * N).reshape(B * K)
    sel_rows = _sc_gather(new_hidden.reshape(B * N, D), gidx2).reshape(B, K, D)

    out_hid, out_act = pl.pallas_call(
        _k7_body,
        grid=(B, nchunks),
        in_specs=[
            pl.BlockSpec((1, K, 1), lambda b, c: (b, 0, 0)),
            pl.BlockSpec((1, K, 1), lambda b, c: (b, 0, 0)),
            pl.BlockSpec((1, K, 1), lambda b, c: (b, 0, 0)),
            pl.BlockSpec((1, K, D), lambda b, c: (b, 0, 0)),
        ],
        out_specs=[
            pl.BlockSpec((1, CHUNK, D), lambda b, c: (b, c, 0)),
            pl.BlockSpec((1, CHUNK, 1), lambda b, c: (b, c, 0)),
        ],
        out_shape=[
            jax.ShapeDtypeStruct((B, N, D), f32),
            jax.ShapeDtypeStruct((B, N, 1), f32),
        ],
    )(idx2, valm, msk, sel_rows)

    return out_act.reshape(B, N), out_hid


# speedup vs baseline: 3.3514x; 1.0869x over previous
import functools

import jax
import jax.numpy as jnp
from jax import lax
from jax.experimental import pallas as pl
from jax.experimental.pallas import tpu as pltpu
from jax.experimental.pallas import tpu_sc as plsc

B, N, D, H = 16, 16384, 128, 4
K = 256
R = 128
C_ = 128
CHUNK = 4096
NSC = 32


def _gelu(x):
    return 0.5 * x * (1.0 + lax.erf(x * 0.7071067811865476))


def _select_topk(a):
    bits = lax.bitcast_convert_type(a, jnp.int32)

    def step(_, lohi):
        lo, hi = lohi
        mid = lo + (hi - lo + 1) // 2
        c = jnp.sum((bits >= mid).astype(jnp.int32))
        ok = c >= K
        return jnp.where(ok, mid, lo), jnp.where(ok, hi, mid - 1)

    T, _ = lax.fori_loop(0, 31, step, (jnp.int32(0), jnp.int32(2**30)))

    maskG = (bits > T)
    nG = jnp.sum(maskG.astype(jnp.int32))
    maskE = (bits == T)
    r_fill = (K - nG).astype(jnp.float32)

    U = (lax.broadcasted_iota(jnp.int32, (R, R), 0)
         < lax.broadcasted_iota(jnp.int32, (R, R), 1)).astype(jnp.float32)
    Ls = U.T

    eM = maskE.astype(jnp.float32)
    prefE = jnp.dot(eM, U, preferred_element_type=jnp.float32)
    offE = jnp.dot(Ls, jnp.sum(eM, axis=1, keepdims=True),
                   preferred_element_type=jnp.float32)
    selE = jnp.logical_and(maskE, (offE + prefE) < r_fill)

    sM = jnp.logical_or(maskG, selE).astype(jnp.float32)
    prefS = jnp.dot(sM, U, preferred_element_type=jnp.float32)
    offS = jnp.dot(Ls, jnp.sum(sM, axis=1, keepdims=True),
                   preferred_element_type=jnp.float32)

    sIota = lax.broadcasted_iota(jnp.int32, (1, K), 1).astype(jnp.float32)
    rIota = lax.broadcasted_iota(jnp.int32, (1, R), 1).astype(jnp.float32)

    Mrs = (offS <= sIota).astype(jnp.float32)
    Rcol = (jnp.sum(Mrs, axis=0, keepdims=True) - 1.0).T
    offR = jnp.max(offS * Mrs, axis=0, keepdims=True)
    jcol = (sIota - offR).T

    OHT = (Rcol == rIota).astype(jnp.float32)
    prefRow = jnp.dot(OHT, prefS, preferred_element_type=jnp.float32)
    selRow = jnp.dot(OHT, sM, preferred_element_type=jnp.float32)
    aRow = jnp.dot(OHT, a, preferred_element_type=jnp.float32)

    OHc = ((prefRow == jcol) & (selRow > 0.5)).astype(jnp.float32)
    Ccol = jnp.sum(OHc * rIota, axis=1, keepdims=True)
    val = jnp.sum(OHc * aRow, axis=1, keepdims=True)
    idxf = Rcol * float(C_) + Ccol
    return idxf, val


def _k1_body(act_ref, idx_ref, val_ref, cnt_ref):
    a = act_ref[0]
    idxf, val = _select_topk(a)
    idx_ref[0] = idxf
    val_ref[0] = val
    cnt = jnp.sum((a > 0.01).astype(jnp.float32))
    cnt_ref[0] = jnp.zeros((K, 1), jnp.float32) + cnt


def _k5_body(sk_ref, act_ref, idx_ref, valm_ref, msk_ref):
    a = act_ref[0]
    idxf, val = _select_topk(a)
    vrow = val.T
    irow = idxf.T
    beats = ((vrow > val) | ((vrow == val) & (irow < idxf))).astype(jnp.float32)
    rank = jnp.sum(beats, axis=1, keepdims=True)
    skf = jnp.minimum(sk_ref[0], N).astype(jnp.float32)
    msk = (rank < skf).astype(jnp.float32)
    idx_ref[0] = idxf
    valm_ref[0] = val * msk
    msk_ref[0] = msk


def _k3_body(km_ref, x_ref, val_ref, idx_ref, wqkv_ref, bqkv_ref, wo_ref, bo_ref,
             msg_ref):
    x = x_ref[0]
    v = val_ref[0]
    idxf = idx_ref[0]
    kmf = km_ref[0] * jnp.float32(1.0)

    vrow = v.T
    irow = idxf.T
    beats = ((vrow > v) | ((vrow == v) & (irow < idxf))).astype(jnp.float32)
    rank = jnp.sum(beats, axis=1, keepdims=True)
    validc = (rank < kmf).astype(jnp.float32)
    valid_row = validc.T

    qkv = jnp.dot(x, wqkv_ref[...], preferred_element_type=jnp.float32) + bqkv_ref[...]
    dh = D // H
    scale = 1.0 / (dh ** 0.5)
    outs = []
    for h in range(H):
        q = qkv[:, h * dh:(h + 1) * dh]
        k = qkv[:, D + h * dh:D + (h + 1) * dh]
        vv = qkv[:, 2 * D + h * dh:2 * D + (h + 1) * dh]
        logits = jnp.dot(q, k.T, preferred_element_type=jnp.float32) * scale
        logits = jnp.where(valid_row > 0.5, logits, -1e30)
        m = jnp.max(logits, axis=1, keepdims=True)
        e = jnp.exp(logits - m)
        attn = e / jnp.sum(e, axis=1, keepdims=True)
        outs.append(jnp.dot(attn, vv, preferred_element_type=jnp.float32))
    o = jnp.concatenate(outs, axis=1)
    o = jnp.dot(o, wo_ref[...], preferred_element_type=jnp.float32) + bo_ref[...]
    msg_ref[0] = o * v * validc


def _k4_body(hid_ref, act_ref, idx_ref, msg_ref,
             w1a_ref, w1b_ref, b1_ref, w2_ref, b2_ref,
             a1a_ref, a1b_ref, ab1_ref, a2_ref, ab2_ref,
             lng_ref, lnb_ref, nh_ref, na_ref):
    c = pl.program_id(1)
    base = (c * CHUNK).astype(jnp.float32)
    hid = hid_ref[0]
    nIota = lax.broadcasted_iota(jnp.int32, (CHUNK, 1), 0).astype(jnp.float32)
    idxrow = idx_ref[0].T
    onehot = ((nIota + base) == idxrow).astype(jnp.float32)
    msgc = jnp.dot(onehot, msg_ref[0], preferred_element_type=jnp.float32)

    h = _gelu(jnp.dot(hid, w1a_ref[...], preferred_element_type=jnp.float32)
              + jnp.dot(msgc, w1b_ref[...], preferred_element_type=jnp.float32)
              + b1_ref[...])
    y = jnp.dot(h, w2_ref[...], preferred_element_type=jnp.float32) + b2_ref[...]
    mu = jnp.mean(y, axis=1, keepdims=True)
    yc = y - mu
    var = jnp.mean(yc * yc, axis=1, keepdims=True)
    nh = yc * lax.rsqrt(var + 1e-5) * lng_ref[...] + lnb_ref[...]

    a2 = _gelu(jnp.dot(hid, a1a_ref[...], preferred_element_type=jnp.float32)
               + jnp.dot(nh, a1b_ref[...], preferred_element_type=jnp.float32)
               + ab1_ref[...])
    dpre = jnp.dot(a2, a2_ref[...], preferred_element_type=jnp.float32) + ab2_ref[...]
    delta = jax.nn.sigmoid(dpre)
    na = jnp.clip(0.7 * act_ref[0] + 0.3 * delta, 0.0, 1.0)
    nh_ref[0] = nh
    na_ref[0] = na


def _k7_body(idx_ref, valm_ref, msk_ref, rows_ref, oh_ref, oa_ref):
    c = pl.program_id(1)
    base = (c * CHUNK).astype(jnp.float32)
    nIota = lax.broadcasted_iota(jnp.int32, (CHUNK, 1), 0).astype(jnp.float32)
    idxrow = idx_ref[0].T
    onehot = ((nIota + base) == idxrow).astype(jnp.float32)
    rows = (rows_ref[0] * msk_ref[0]).astype(jnp.bfloat16)
    ohb = onehot.astype(jnp.bfloat16)
    oh_ref[0] = jnp.dot(ohb, rows, preferred_element_type=jnp.float32)
    oa_ref[0] = jnp.dot(ohb, valm_ref[0].astype(jnp.bfloat16),
                        preferred_element_type=jnp.float32)


def _sc_gather(table, idx_flat):
    nidx = idx_flat.shape[0]
    per_w = nidx // NSC
    mesh = plsc.VectorSubcoreMesh(core_axis_name="c", subcore_axis_name="s")

    @functools.partial(
        pl.kernel, mesh=mesh,
        out_type=jax.ShapeDtypeStruct((nidx, D), jnp.float32),
        scratch_types=[
            pltpu.VMEM((per_w,), jnp.int32),
            pltpu.VMEM((per_w, D), jnp.float32),
            pltpu.SemaphoreType.DMA,
        ],
    )
    def gath(table_hbm, idx_hbm, out_hbm, idx_v, rows_v, sem):
        wid = lax.axis_index("s") * 2 + lax.axis_index("c")
        base = wid * per_w
        pltpu.sync_copy(idx_hbm.at[pl.ds(base, per_w)], idx_v)
        pltpu.async_copy(table_hbm.at[idx_v], rows_v, sem).wait()
        pltpu.sync_copy(rows_v, out_hbm.at[pl.ds(base, per_w)])

    return gath(table, idx_flat)


def kernel(activation, hidden_state, sparsity_k, in_proj_w, in_proj_b, out_w, out_b,
           su_w1, su_b1, su_w2, su_b2, au_w1, au_b1, au_w2, au_b2, ln_g, ln_b):
    f32 = jnp.float32
    act_sq = activation.reshape(B, R, C_)
    act_col = activation.reshape(B, N, 1)

    idx1, val1, cnt = pl.pallas_call(
        _k1_body,
        grid=(B,),
        in_specs=[pl.BlockSpec((1, R, C_), lambda b: (b, 0, 0))],
        out_specs=[pl.BlockSpec((1, K, 1), lambda b: (b, 0, 0))] * 3,
        out_shape=[jax.ShapeDtypeStruct((B, K, 1), f32)] * 3,
    )(act_sq)

    k_msg = jnp.maximum(jnp.minimum(jnp.max(cnt[:, 0, 0]), 256.0), 1.0)
    k_msg_arr = k_msg.astype(jnp.int32).reshape(1)

    gidx1 = (idx1[:, :, 0].astype(jnp.int32)
             + jnp.arange(B, dtype=jnp.int32)[:, None] * N).reshape(B * K)
    active = _sc_gather(hidden_state.reshape(B * N, D), gidx1).reshape(B, K, D)

    wqkv_t = in_proj_w.T
    bqkv_r = in_proj_b.reshape(1, 3 * D)
    wo_t = out_w.T
    bo_r = out_b.reshape(1, D)
    msg = pl.pallas_call(
        _k3_body,
        grid=(B,),
        in_specs=[
            pl.BlockSpec(memory_space=pltpu.SMEM),
            pl.BlockSpec((1, K, D), lambda b: (b, 0, 0)),
            pl.BlockSpec((1, K, 1), lambda b: (b, 0, 0)),
            pl.BlockSpec((1, K, 1), lambda b: (b, 0, 0)),
            pl.BlockSpec((D, 3 * D), lambda b: (0, 0)),
            pl.BlockSpec((1, 3 * D), lambda b: (0, 0)),
            pl.BlockSpec((D, D), lambda b: (0, 0)),
            pl.BlockSpec((1, D), lambda b: (0, 0)),
        ],
        out_specs=pl.BlockSpec((1, K, D), lambda b: (b, 0, 0)),
        out_shape=jax.ShapeDtypeStruct((B, K, D), f32),
    )(k_msg_arr, active, val1, idx1, wqkv_t, bqkv_r, wo_t, bo_r)

    w1a = su_w1[:, :D].T
    w1b = su_w1[:, D:].T
    b1 = su_b1.reshape(1, D)
    w2 = su_w2.T
    b2 = su_b2.reshape(1, D)
    a1a = au_w1[:, :D].T
    a1b = au_w1[:, D:].T
    ab1 = au_b1.reshape(1, D)
    a2w = au_w2.T
    ab2 = au_b2.reshape(1, 1)
    lng = ln_g.reshape(1, D)
    lnb = ln_b.reshape(1, D)

    nchunks = N // CHUNK
    new_hidden, new_act = pl.pallas_call(
        _k4_body,
        grid=(B, nchunks),
        in_specs=[
            pl.BlockSpec((1, CHUNK, D), lambda b, c: (b, c, 0)),
            pl.BlockSpec((1, CHUNK, 1), lambda b, c: (b, c, 0)),
            pl.BlockSpec((1, K, 1), lambda b, c: (b, 0, 0)),
            pl.BlockSpec((1, K, D), lambda b, c: (b, 0, 0)),
            pl.BlockSpec((D, D), lambda b, c: (0, 0)),
            pl.BlockSpec((D, D), lambda b, c: (0, 0)),
            pl.BlockSpec((1, D), lambda b, c: (0, 0)),
            pl.BlockSpec((D, D), lambda b, c: (0, 0)),
            pl.BlockSpec((1, D), lambda b, c: (0, 0)),
            pl.BlockSpec((D, D), lambda b, c: (0, 0)),
            pl.BlockSpec((D, D), lambda b, c: (0, 0)),
            pl.BlockSpec((1, D), lambda b, c: (0, 0)),
            pl.BlockSpec((D, 1), lambda b, c: (0, 0)),
            pl.BlockSpec((1, 1), lambda b, c: (0, 0)),
            pl.BlockSpec((1, D), lambda b, c: (0, 0)),
            pl.BlockSpec((1, D), lambda b, c: (0, 0)),
        ],
        out_specs=[
            pl.BlockSpec((1, CHUNK, D), lambda b, c: (b, c, 0)),
            pl.BlockSpec((1, CHUNK, 1), lambda b, c: (b, c, 0)),
        ],
        out_shape=[
            jax.ShapeDtypeStruct((B, N, D), f32),
            jax.ShapeDtypeStruct((B, N, 1), f32),
        ],
    )(hidden_state, act_col, idx1, msg,
      w1a, w1b, b1, w2, b2, a1a, a1b, ab1, a2w, ab2, lng, lnb)

    sk_arr = jnp.asarray(sparsity_k, jnp.int32).reshape(1)
    idx2, valm, msk = pl.pallas_call(
        _k5_body,
        grid=(B,),
        in_specs=[
            pl.BlockSpec(memory_space=pltpu.SMEM),
            pl.BlockSpec((1, R, C_), lambda b: (b, 0, 0)),
        ],
        out_specs=[pl.BlockSpec((1, K, 1), lambda b: (b, 0, 0))] * 3,
        out_shape=[jax.ShapeDtypeStruct((B, K, 1), f32)] * 3,
    )(sk_arr, new_act.reshape(B, R, C_))

    gidx2 = (idx2[:, :, 0].astype(jnp.int32)
             + jnp.arange(B, dtype=jnp.int32)[:, None] * N).reshape(B * K)
    sel_rows = _sc_gather(new_hidden.reshape(B * N, D), gidx2).reshape(B, K, D)

    out_hid, out_act = pl.pallas_call(
        _k7_body,
        grid=(B, nchunks),
        in_specs=[
            pl.BlockSpec((1, K, 1), lambda b, c: (b, 0, 0)),
            pl.BlockSpec((1, K, 1), lambda b, c: (b, 0, 0)),
            pl.BlockSpec((1, K, 1), lambda b, c: (b, 0, 0)),
            pl.BlockSpec((1, K, D), lambda b, c: (b, 0, 0)),
        ],
        out_specs=[
            pl.BlockSpec((1, CHUNK, D), lambda b, c: (b, c, 0)),
            pl.BlockSpec((1, CHUNK, 1), lambda b, c: (b, c, 0)),
        ],
        out_shape=[
            jax.ShapeDtypeStruct((B, N, D), f32),
            jax.ShapeDtypeStruct((B, N, 1), f32),
        ],
    )(idx2, valm, msk, sel_rows)

    return out_act.reshape(B, N), out_hid
